# Initial kernel scaffold; baseline (speedup 1.0000x reference)
import jax
import jax.numpy as jnp
from jax.experimental import pallas as pl


def kernel(x, edge_index, edge_weight, W1, b1, W2, b2):
    raise NotImplementedError("write your pallas kernel here")


# speedup vs baseline: 4.2265x; 4.2265x over previous
import jax
import jax.numpy as jnp
from jax import lax
from jax.experimental import pallas as pl
from jax.experimental.pallas import tpu as pltpu
from jax.experimental.pallas import tpu_sc as plsc

N = 10000
F1 = 128
F2 = 64
E = 320000

NC = 2
NS = 16
NW = NC * NS
EW = E // NW
K = 80
NCHUNK = EW // K
NP = 10240
ROWS_PER_SUB = NP // NS


def _make_spmm(F):
    mesh = plsc.VectorSubcoreMesh(core_axis_name="c", subcore_axis_name="s")

    def body(src_hbm, dst_hbm, w_hbm, sup_hbm, zero_hbm, out_hbm,
             src_v, dst_v, w_v, rows_v, acc, sem):
        cid = lax.axis_index("c")
        sid = lax.axis_index("s")
        wid = sid * NC + cid
        base = wid * EW

        pltpu.sync_copy(zero_hbm.at[pl.ds(sid * ROWS_PER_SUB, ROWS_PER_SUB)],
                        acc.at[pl.ds(sid * ROWS_PER_SUB, ROWS_PER_SUB)])
        plsc.subcore_barrier()

        def chunk(i, carry):
            off = base + i * K
            pltpu.sync_copy(src_hbm.at[pl.ds(off, K)], src_v)
            pltpu.sync_copy(dst_hbm.at[pl.ds(off, K)], dst_v)
            pltpu.sync_copy(w_hbm.at[pl.ds(off, K)], w_v)
            pltpu.async_copy(sup_hbm.at[src_v], rows_v, sem).wait()

            def group(g, c2):
                wvec = w_v[pl.ds(g * 16, 16)]
                for i in range(16):
                    e = g * 16 + i
                    w = wvec[i]
                    for j in range(F // 16):
                        sl = pl.ds(j * 16, 16)
                        rows_v[e, sl] = rows_v[e, sl] * w
                return c2
            lax.fori_loop(0, K // 16, group, 0)
            pltpu.sync_copy(rows_v, acc.at[dst_v], add=True)
            return carry
        lax.fori_loop(0, NCHUNK, chunk, 0)

        plsc.subcore_barrier()
        pltpu.sync_copy(acc.at[pl.ds(sid * ROWS_PER_SUB, ROWS_PER_SUB)],
                        out_hbm.at[cid, pl.ds(sid * ROWS_PER_SUB, ROWS_PER_SUB)])

    return pl.kernel(
        body,
        out_type=jax.ShapeDtypeStruct((NC, NP, F), jnp.float32),
        mesh=mesh,
        scratch_types=[
            pltpu.VMEM((K,), jnp.int32),
            pltpu.VMEM((K,), jnp.int32),
            pltpu.VMEM((K,), jnp.float32),
            pltpu.VMEM((K, F), jnp.float32),
            pltpu.VMEM_SHARED((NP, F), jnp.float32),
            pltpu.SemaphoreType.DMA,
        ],
        compiler_params=pltpu.CompilerParams(use_tc_tiling_on_sc=False),
    )


_spmm1 = _make_spmm(F1)
_spmm2 = _make_spmm(F2)


def _mm_body(x_ref, w_ref, o_ref):
    o_ref[...] = jnp.dot(x_ref[...], w_ref[...],
                         preferred_element_type=jnp.float32)


def _tc_mm(x, w):
    return pl.pallas_call(
        _mm_body,
        out_shape=jax.ShapeDtypeStruct((x.shape[0], w.shape[1]), jnp.float32),
    )(x, w)


def _mid_body(p_ref, b1_ref, w2_ref, o_ref):
    h = p_ref[0] + p_ref[1] + b1_ref[...]
    h = jnp.maximum(h, 0.0)
    o_ref[...] = jnp.dot(h, w2_ref[...], preferred_element_type=jnp.float32)


def _tc_mid(p, b1, w2):
    return pl.pallas_call(
        _mid_body,
        out_shape=jax.ShapeDtypeStruct((N, F2), jnp.float32),
    )(p, b1, w2)


def _out_body(p_ref, b2_ref, o_ref):
    z = p_ref[0] + p_ref[1] + b2_ref[...]
    m = jnp.max(z, axis=1, keepdims=True)
    zs = z - m
    o_ref[...] = zs - jnp.log(jnp.sum(jnp.exp(zs), axis=1, keepdims=True))


def _tc_out(p, b2):
    return pl.pallas_call(
        _out_body,
        out_shape=jax.ShapeDtypeStruct((N, F2), jnp.float32),
    )(p, b2)


@jax.jit
def kernel(x, edge_index, edge_weight, W1, b1, W2, b2):
    src = edge_index[1]
    dst = edge_index[0]
    zeros1 = jnp.zeros((NP, F1), jnp.float32)
    zeros2 = jnp.zeros((NP, F2), jnp.float32)

    support = _tc_mm(x, W1)
    p1 = _spmm1(src, dst, edge_weight, support, zeros1)
    s2 = _tc_mid(p1[:, :N], b1.reshape(1, F1), W2)
    p2 = _spmm2(src, dst, edge_weight, s2, zeros2)
    return _tc_out(p2[:, :N], b2.reshape(1, F2))


# speedup vs baseline: 5.0017x; 1.1834x over previous
import jax
import jax.numpy as jnp
from jax import lax
from jax.experimental import pallas as pl
from jax.experimental.pallas import tpu as pltpu
from jax.experimental.pallas import tpu_sc as plsc

N = 10000
F1 = 128
F2 = 64
E = 320000

NC = 2
NS = 16
NW = NC * NS
EW = E // NW
K = 80
NCHUNK = EW // K
NP = 10240
ROWS_PER_SUB = NP // NS


NB = 5
NBLK = NCHUNK // NB
FW = 64


def _make_spmm(P):
    mesh = plsc.VectorSubcoreMesh(core_axis_name="c", subcore_axis_name="s")

    def body(idx_hbm, dst_hbm, w_hbm, tab_hbm, out_hbm,
             src_all, dst_all, w_all,
             rows0, rows1, rows2, rows3, rows4, acc,
             g0, g1, g2, g3, g4, s0, s1, s2, s3, s4):
        rows = [rows0, rows1, rows2, rows3, rows4]
        gsem = [g0, g1, g2, g3, g4]
        ssem = [s0, s1, s2, s3, s4]
        cid = lax.axis_index("c")
        sid = lax.axis_index("s")
        wid = sid * NC + cid

        pltpu.sync_copy(dst_hbm.at[wid], dst_all)
        pltpu.sync_copy(w_hbm.at[wid], w_all)

        def gidx(c):
            return src_all.at[pl.ds(c * K, K)]

        def scale(c, b):
            def group(g, c2):
                wvec = w_all[pl.ds(c * K + g * 16, 16)]
                for t in range(16):
                    e = g * 16 + t
                    wv = wvec[t]
                    for j in range(FW // 16):
                        sl = pl.ds(j * 16, 16)
                        rows[b][e, sl] = rows[b][e, sl] * wv
                return c2
            lax.fori_loop(0, K // 16, group, 0)

        for p in range(P):
            pltpu.sync_copy(idx_hbm.at[p, wid], src_all)

            def zrow(r, c):
                for j in range(FW // 16):
                    rows0[r, pl.ds(j * 16, 16)] = jnp.zeros((16,), jnp.float32)
                return c
            lax.fori_loop(0, K, zrow, 0)
            for t in range(ROWS_PER_SUB // K):
                pltpu.sync_copy(
                    rows0, acc.at[pl.ds(sid * ROWS_PER_SUB + t * K, K)])
            plsc.subcore_barrier()

            for b in range(NB):
                pltpu.async_copy(tab_hbm.at[gidx(b)], rows[b], gsem[b])

            def blk(i, carry):
                for b in range(NB):
                    c = i * NB + b
                    pltpu.make_async_copy(tab_hbm.at[gidx(c)], rows[b],
                                          gsem[b]).wait()
                    scale(c, b)
                    pltpu.async_copy(rows[b], acc.at[dst_all.at[c]], ssem[b],
                                     add=True)

                @pl.when(i < NBLK - 1)
                def _():
                    for b in range(NB):
                        cn = (i + 1) * NB + b
                        pltpu.make_async_copy(rows[b], acc.at[dst_all.at[cn]],
                                              ssem[b]).wait()
                        pltpu.async_copy(tab_hbm.at[gidx(cn)], rows[b], gsem[b])
                return carry
            lax.fori_loop(0, NBLK, blk, 0)

            for b in range(NB):
                pltpu.make_async_copy(
                    rows[b], acc.at[dst_all.at[NCHUNK - NB + b]],
                    ssem[b]).wait()
            plsc.subcore_barrier()
            pltpu.sync_copy(
                acc.at[pl.ds(sid * ROWS_PER_SUB, ROWS_PER_SUB)],
                out_hbm.at[p, cid, pl.ds(sid * ROWS_PER_SUB, ROWS_PER_SUB)])
            if p + 1 < P:
                plsc.subcore_barrier()

    return pl.kernel(
        body,
        out_type=jax.ShapeDtypeStruct((P, NC, NP, FW), jnp.float32),
        mesh=mesh,
        scratch_types=[
            pltpu.VMEM((EW,), jnp.int32),
            pltpu.VMEM((NCHUNK, K), jnp.int32),
            pltpu.VMEM((EW,), jnp.float32),
        ] + [pltpu.VMEM((K, FW), jnp.float32)] * NB + [
            pltpu.VMEM_SHARED((NP, FW), jnp.float32),
        ] + [pltpu.SemaphoreType.DMA] * (2 * NB),
        compiler_params=pltpu.CompilerParams(use_tc_tiling_on_sc=False),
    )


_spmm1 = _make_spmm(2)
_spmm2 = _make_spmm(1)


def _mm_body(x_ref, w_ref, o_ref):
    o_ref[...] = jnp.dot(x_ref[...], w_ref[...],
                         preferred_element_type=jnp.float32)


def _tc_mm(x, w):
    return pl.pallas_call(
        _mm_body,
        out_shape=jax.ShapeDtypeStruct((x.shape[0], w.shape[1]), jnp.float32),
    )(x, w)


def _mid_body(p_ref, b1_ref, w2_ref, o_ref):
    pv = p_ref[...]
    h0 = pv[0, 0, :N] + pv[0, 1, :N]
    h1 = pv[1, 0, :N] + pv[1, 1, :N]
    h = jnp.concatenate([h0, h1], axis=1) + b1_ref[...]
    h = jnp.maximum(h, 0.0)
    o_ref[...] = jnp.dot(h, w2_ref[...], preferred_element_type=jnp.float32)


def _tc_mid(p, b1, w2):
    return pl.pallas_call(
        _mid_body,
        out_shape=jax.ShapeDtypeStruct((N, F2), jnp.float32),
    )(p, b1, w2)


def _out_body(p_ref, b2_ref, o_ref):
    pv = p_ref[...]
    z = pv[0, :N] + pv[1, :N] + b2_ref[...]
    m = jnp.max(z, axis=1, keepdims=True)
    zs = z - m
    o_ref[...] = zs - jnp.log(jnp.sum(jnp.exp(zs), axis=1, keepdims=True))


def _tc_out(p, b2):
    return pl.pallas_call(
        _out_body,
        out_shape=jax.ShapeDtypeStruct((N, F2), jnp.float32),
    )(p, b2)


@jax.jit
def kernel(x, edge_index, edge_weight, W1, b1, W2, b2):
    src = edge_index[1].reshape(NW, EW)
    dst = edge_index[0].reshape(NW, NCHUNK, K)
    w = edge_weight.reshape(NW, EW)
    idx1 = jnp.stack([2 * src, 2 * src + 1])
    idx2 = src[None]

    support = _tc_mm(x, W1)
    tab1 = support.reshape(2 * N, FW)
    p1 = _spmm1(idx1, dst, w, tab1)
    s2 = _tc_mid(p1, b1.reshape(1, F1), W2)
    p2 = _spmm2(idx2, dst, w, s2)
    return _tc_out(p2[0], b2.reshape(1, F2))


# speedup vs baseline: 11.4182x; 2.2829x over previous
import jax
import jax.numpy as jnp
from jax import lax
from jax.experimental import pallas as pl
from jax.experimental.pallas import tpu as pltpu
from jax.experimental.pallas import tpu_sc as plsc

N = 10000
F1 = 128
F2 = 64
E = 320000

NC = 2
NS = 16
NW = NC * NS
EW = E // NW
K = 80
NCHUNK = EW // K
NP = 10240
ROWS_PER_SUB = NP // NS


NB = 5
NBLK = NCHUNK // NB
FW = 64


def _make_spmm(P):
    mesh = plsc.VectorSubcoreMesh(core_axis_name="c", subcore_axis_name="s")

    def body(idx_hbm, dst_hbm, w_hbm, tab_hbm, out_hbm,
             src_all, dst_all, w_all,
             rows0, rows1, rows2, rows3, rows4,
             sc0, sc1, sc2, sc3, sc4, acc,
             g0, g1, g2, g3, g4, s0, s1, s2, s3, s4):
        rows = [rows0, rows1, rows2, rows3, rows4]
        scl = [sc0, sc1, sc2, sc3, sc4]
        gsem = [g0, g1, g2, g3, g4]
        ssem = [s0, s1, s2, s3, s4]
        cid = lax.axis_index("c")
        sid = lax.axis_index("s")
        wid = sid * NC + cid

        pltpu.sync_copy(dst_hbm.at[wid], dst_all)
        pltpu.sync_copy(w_hbm.at[wid], w_all)

        def gidx(c):
            return src_all.at[pl.ds(c * K, K)]

        def scale(c, b):
            @plsc.parallel_loop(0, K // 16, unroll=2)
            def group(g):
                wvec = w_all[pl.ds(c * K + g * 16, 16)]
                for t in range(16):
                    e = g * 16 + t
                    wv = wvec[t]
                    for j in range(FW // 16):
                        sl = pl.ds(j * 16, 16)
                        scl[b][e, sl] = rows[b][e, sl] * wv

        for p in range(P):
            pltpu.sync_copy(idx_hbm.at[p, wid], src_all)

            def zrow(r, c):
                for j in range(FW // 16):
                    rows0[r, pl.ds(j * 16, 16)] = jnp.zeros((16,), jnp.float32)
                return c
            lax.fori_loop(0, K, zrow, 0)
            for t in range(ROWS_PER_SUB // K):
                pltpu.sync_copy(
                    rows0, acc.at[pl.ds(sid * ROWS_PER_SUB + t * K, K)])
            plsc.subcore_barrier()

            for b in range(NB):
                pltpu.async_copy(tab_hbm.at[gidx(b)], rows[b], gsem[b])

            def blk(i, carry):
                for b in range(NB):
                    c = i * NB + b
                    pltpu.make_async_copy(tab_hbm.at[gidx(c)], rows[b],
                                          gsem[b]).wait()
                    scale(c, b)
                    pltpu.async_copy(scl[b], acc.at[dst_all.at[c]], ssem[b],
                                     add=True)

                @pl.when(i < NBLK - 1)
                def _():
                    for b in range(NB):
                        cn = (i + 1) * NB + b
                        pltpu.make_async_copy(scl[b], acc.at[dst_all.at[cn]],
                                              ssem[b]).wait()
                        pltpu.async_copy(tab_hbm.at[gidx(cn)], rows[b], gsem[b])
                return carry
            lax.fori_loop(0, NBLK, blk, 0)

            for b in range(NB):
                pltpu.make_async_copy(
                    scl[b], acc.at[dst_all.at[NCHUNK - NB + b]],
                    ssem[b]).wait()
            plsc.subcore_barrier()
            pltpu.sync_copy(
                acc.at[pl.ds(sid * ROWS_PER_SUB, ROWS_PER_SUB)],
                out_hbm.at[p, cid, pl.ds(sid * ROWS_PER_SUB, ROWS_PER_SUB)])
            if p + 1 < P:
                plsc.subcore_barrier()

    return pl.kernel(
        body,
        out_type=jax.ShapeDtypeStruct((P, NC, NP, FW), jnp.float32),
        mesh=mesh,
        scratch_types=[
            pltpu.VMEM((EW,), jnp.int32),
            pltpu.VMEM((NCHUNK, K), jnp.int32),
            pltpu.VMEM((EW,), jnp.float32),
        ] + [pltpu.VMEM((K, FW), jnp.float32)] * (2 * NB) + [
            pltpu.VMEM_SHARED((NP, FW), jnp.float32),
        ] + [pltpu.SemaphoreType.DMA] * (2 * NB),
        compiler_params=pltpu.CompilerParams(use_tc_tiling_on_sc=False),
    )


_spmm1 = _make_spmm(2)
_spmm2 = _make_spmm(1)


def _mm_body(x_ref, w_ref, o_ref):
    o_ref[...] = jnp.dot(x_ref[...], w_ref[...],
                         preferred_element_type=jnp.float32)


def _tc_mm(x, w):
    return pl.pallas_call(
        _mm_body,
        out_shape=jax.ShapeDtypeStruct((x.shape[0], w.shape[1]), jnp.float32),
    )(x, w)


def _mid_body(p_ref, b1_ref, w2_ref, o_ref):
    pv = p_ref[...]
    h0 = pv[0, 0, :N] + pv[0, 1, :N]
    h1 = pv[1, 0, :N] + pv[1, 1, :N]
    h = jnp.concatenate([h0, h1], axis=1) + b1_ref[...]
    h = jnp.maximum(h, 0.0)
    o_ref[...] = jnp.dot(h, w2_ref[...], preferred_element_type=jnp.float32)


def _tc_mid(p, b1, w2):
    return pl.pallas_call(
        _mid_body,
        out_shape=jax.ShapeDtypeStruct((N, F2), jnp.float32),
    )(p, b1, w2)


def _out_body(p_ref, b2_ref, o_ref):
    pv = p_ref[...]
    z = pv[0, :N] + pv[1, :N] + b2_ref[...]
    m = jnp.max(z, axis=1, keepdims=True)
    zs = z - m
    o_ref[...] = zs - jnp.log(jnp.sum(jnp.exp(zs), axis=1, keepdims=True))


def _tc_out(p, b2):
    return pl.pallas_call(
        _out_body,
        out_shape=jax.ShapeDtypeStruct((N, F2), jnp.float32),
    )(p, b2)


@jax.jit
def kernel(x, edge_index, edge_weight, W1, b1, W2, b2):
    src = edge_index[1].reshape(NW, EW)
    dst = edge_index[0].reshape(NW, NCHUNK, K)
    w = edge_weight.reshape(NW, EW)
    idx1 = jnp.stack([2 * src, 2 * src + 1])
    idx2 = src[None]

    support = _tc_mm(x, W1)
    tab1 = support.reshape(2 * N, FW)
    p1 = _spmm1(idx1, dst, w, tab1)
    s2 = _tc_mid(p1, b1.reshape(1, F1), W2)
    p2 = _spmm2(idx2, dst, w, s2)
    return _tc_out(p2[0], b2.reshape(1, F2))
